# baseline (speedup 1.0000x reference)
import functools
import math

import jax
import jax.numpy as jnp
from jax import lax
from jax.experimental import pallas as pl
from jax.experimental.pallas import tpu as pltpu
from jax.experimental.pallas import tpu_sc as plsc

VOCAB = 1000000
DIMS = 64
MAX_TOK = 200
BATCH = 4096
LANES = 16

NC, NS = 2, 16
NW = NC * NS
BBLK = 128
NCHUNK = MAX_TOK
DH, DL = DIMS // 8, 8
BQ = BBLK // LANES
SG = 4
NSG = NCHUNK // SG
NO = 2


def _pe_table():
    position = jnp.arange(0, MAX_TOK, dtype=jnp.float32)[:, None]
    div_term = jnp.exp(
        jnp.arange(0, DIMS, 2, dtype=jnp.float32) * -(math.log(10000.0) / DIMS))
    pe = jnp.zeros((MAX_TOK, DIMS), dtype=jnp.float32)
    pe = pe.at[:, 0::2].set(jnp.sin(position * div_term))
    pe = pe.at[:, 1::2].set(jnp.cos(position * div_term))
    return pe


_mesh = plsc.VectorSubcoreMesh(core_axis_name="c", subcore_axis_name="s")


@functools.partial(
    pl.kernel,
    out_type=jax.ShapeDtypeStruct((MAX_TOK, DH, NW, DL, BBLK), jnp.float32),
    mesh=_mesh,
    compiler_params=pltpu.CompilerParams(
        needs_layout_passes=False, use_tc_tiling_on_sc=False),
    scratch_types=(
        [pltpu.VMEM((NSG, SG * BBLK), jnp.int32),
         pltpu.VMEM((MAX_TOK, DIMS), jnp.float32),
         pltpu.VMEM((DIMS + LANES,), jnp.float32)]
        + [pltpu.VMEM((SG * BBLK, DIMS), jnp.float32)] * 2
        + [pltpu.VMEM((DH, DL, BBLK), jnp.float32)] * NO
        + [pltpu.SemaphoreType.DMA] * (2 + NO)
    ),
)
def _emb_lookup(x_hbm, pe_hbm, table_hbm, out_hbm,
                idx_v, pe_v, pe_t, *bufs):
    gb, ob = bufs[:2], bufs[2:2 + NO]
    gs, os_ = bufs[2 + NO:4 + NO], bufs[4 + NO:]
    wid = lax.axis_index("s") * NC + lax.axis_index("c")
    pltpu.sync_copy(x_hbm.at[wid], idx_v)
    pltpu.sync_copy(pe_hbm, pe_v)

    lane = lax.iota(jnp.int32, LANES)
    row_sel = tuple(bq * LANES + lane for bq in range(BQ))

    def start_gather(sg, p):
        pltpu.async_copy(table_hbm.at[idx_v.at[sg]], gb[p], gs[p])

    def wait_gather(sg, p):
        pltpu.make_async_copy(
            table_hbm.at[idx_v.at[sg]], gb[p], gs[p]).wait()

    def writeout(t, o, wait):
        cp = pltpu.make_async_copy(ob[o], out_hbm.at[t, :, wid], os_[o])
        if wait:
            cp.wait()
        else:
            cp.start()

    start_gather(0, 0)
    start_gather(1, 1)

    @pl.loop(0, NSG, step=2)
    def _super(s0):
        for p in range(2):
            sg = s0 + p
            wait_gather(sg, p)
            for j in range(SG):
                t = sg * SG + j
                o = j % NO

                @pl.when(t >= NO)
                def _wait_out():
                    writeout(t - NO, o, wait=True)

                for i in range(DIMS // LANES):
                    pe_t[pl.ds(i * LANES, LANES)] = pe_v[
                        t, pl.ds(i * LANES, LANES)]
                pe_t[pl.ds(DIMS, LANES)] = pe_v[t, pl.ds(0, LANES)]

                @pl.loop(0, DIMS, unroll=4)
                def _dloop(d):
                    drot = (d + lane) & (DIMS - 1)
                    dhv = lax.shift_right_logical(drot, 3)
                    dlv = drot & (DL - 1)
                    pr = pe_t[pl.ds(d, LANES)]
                    for bq in range(BQ):
                        val = plsc.load_gather(
                            gb[p], [j * BBLK + row_sel[bq], drot])
                        plsc.store_scatter(
                            ob[o], [dhv, dlv, row_sel[bq]], val + pr)

                zmask = idx_v[sg, pl.ds(j * BBLK, LANES)] == 0
                for q in range(1, BQ):
                    zmask = zmask | (
                        idx_v[sg, pl.ds(j * BBLK + q * LANES, LANES)] == 0)
                n0 = plsc.all_reduce_population_count(zmask)[0]

                @pl.when(n0 > 0)
                def _fixup():
                    for bq in range(BQ):
                        mf = jnp.where(
                            idx_v[sg,
                                  pl.ds(j * BBLK + bq * LANES, LANES)] == 0,
                            jnp.float32(0.0), jnp.float32(1.0))

                        @pl.loop(0, DIMS)
                        def _dfix(d):
                            sl = pl.ds(bq * LANES, LANES)
                            dh, dl = d // DL, d % DL
                            pez = plsc.load_gather(
                                pe_t, [jnp.full((LANES,), d, jnp.int32)])
                            ob[o][dh, dl, sl] = (
                                (ob[o][dh, dl, sl] - pez) * mf + pez)

                writeout(t, o, wait=False)

            @pl.when(s0 + 2 < NSG)
            def _next_gather():
                start_gather(sg + 2, p)

    for j in range(NO):
        writeout(NCHUNK - NO + j, (NCHUNK - NO + j) % NO, wait=True)


def kernel(x, table):
    xr = (x.T.reshape(MAX_TOK, NW, BBLK).transpose(1, 0, 2)
          .reshape(NW, NSG, SG * BBLK).astype(jnp.int32))
    out5 = _emb_lookup(xr, _pe_table(), table)
    return out5.transpose(2, 4, 0, 1, 3).reshape(BATCH, MAX_TOK, DIMS)


# speedup vs baseline: 1.0384x; 1.0384x over previous
import functools
import math

import jax
import jax.numpy as jnp
from jax import lax
from jax.experimental import pallas as pl
from jax.experimental.pallas import tpu as pltpu
from jax.experimental.pallas import tpu_sc as plsc

VOCAB = 1000000
DIMS = 64
MAX_TOK = 200
BATCH = 4096
LANES = 16

NC, NS = 2, 16
NW = NC * NS
BBLK = 128
NCHUNK = MAX_TOK
DH, DL = DIMS // 8, 8
BQ = BBLK // LANES
SG = 4
NSG = NCHUNK // SG
NO = 2


def _pe_table():
    position = jnp.arange(0, MAX_TOK, dtype=jnp.float32)[:, None]
    div_term = jnp.exp(
        jnp.arange(0, DIMS, 2, dtype=jnp.float32) * -(math.log(10000.0) / DIMS))
    pe = jnp.zeros((MAX_TOK, DIMS), dtype=jnp.float32)
    pe = pe.at[:, 0::2].set(jnp.sin(position * div_term))
    pe = pe.at[:, 1::2].set(jnp.cos(position * div_term))
    return pe


_mesh = plsc.VectorSubcoreMesh(core_axis_name="c", subcore_axis_name="s")


@functools.partial(
    pl.kernel,
    out_type=jax.ShapeDtypeStruct((MAX_TOK, DH, NW, DL, BBLK), jnp.float32),
    mesh=_mesh,
    compiler_params=pltpu.CompilerParams(
        needs_layout_passes=False, use_tc_tiling_on_sc=False),
    scratch_types=(
        [pltpu.VMEM((NSG, SG * BBLK), jnp.int32),
         pltpu.VMEM((MAX_TOK, DIMS), jnp.float32),
         pltpu.VMEM((DIMS + LANES,), jnp.float32)]
        + [pltpu.VMEM((SG * BBLK, DIMS), jnp.float32)] * 2
        + [pltpu.VMEM((DH, DL, BBLK), jnp.float32)] * NO
        + [pltpu.SemaphoreType.DMA] * (2 + NO)
    ),
)
def _emb_lookup(x_hbm, pe_hbm, table_hbm, out_hbm,
                idx_v, pe_v, pe_t, *bufs):
    gb, ob = bufs[:2], bufs[2:2 + NO]
    gs, os_ = bufs[2 + NO:4 + NO], bufs[4 + NO:]
    wid = lax.axis_index("s") * NC + lax.axis_index("c")
    pltpu.sync_copy(x_hbm.at[wid], idx_v)
    pltpu.sync_copy(pe_hbm, pe_v)

    lane = lax.iota(jnp.int32, LANES)
    row_sel = tuple(bq * LANES + lane for bq in range(BQ))

    def start_gather(sg, p):
        pltpu.async_copy(table_hbm.at[idx_v.at[sg]], gb[p], gs[p])

    def wait_gather(sg, p):
        pltpu.make_async_copy(
            table_hbm.at[idx_v.at[sg]], gb[p], gs[p]).wait()

    def writeout(t, o, wait):
        cp = pltpu.make_async_copy(ob[o], out_hbm.at[t, :, wid], os_[o])
        if wait:
            cp.wait()
        else:
            cp.start()

    start_gather(0, 0)
    start_gather(1, 1)

    @pl.loop(0, NSG, step=2)
    def _super(s0):
        for p in range(2):
            sg = s0 + p
            wait_gather(sg, p)
            for j in range(SG):
                t = sg * SG + j
                o = j % NO

                @pl.when(t >= NO)
                def _wait_out():
                    writeout(t - NO, o, wait=True)

                for i in range(DIMS // LANES):
                    pe_t[pl.ds(i * LANES, LANES)] = pe_v[
                        t, pl.ds(i * LANES, LANES)]
                pe_t[pl.ds(DIMS, LANES)] = pe_v[t, pl.ds(0, LANES)]

                @pl.loop(0, DIMS, unroll=8)
                def _dloop(d):
                    drot = (d + lane) & (DIMS - 1)
                    dhv = lax.shift_right_logical(drot, 3)
                    dlv = drot & (DL - 1)
                    pr = pe_t[pl.ds(d, LANES)]
                    for bq in range(BQ):
                        val = plsc.load_gather(
                            gb[p], [j * BBLK + row_sel[bq], drot])
                        plsc.store_scatter(
                            ob[o], [dhv, dlv, row_sel[bq]], val + pr)

                zmask = idx_v[sg, pl.ds(j * BBLK, LANES)] == 0
                for q in range(1, BQ):
                    zmask = zmask | (
                        idx_v[sg, pl.ds(j * BBLK + q * LANES, LANES)] == 0)
                n0 = plsc.all_reduce_population_count(zmask)[0]

                @pl.when(n0 > 0)
                def _fixup():
                    for bq in range(BQ):
                        mf = jnp.where(
                            idx_v[sg,
                                  pl.ds(j * BBLK + bq * LANES, LANES)] == 0,
                            jnp.float32(0.0), jnp.float32(1.0))

                        @pl.loop(0, DIMS)
                        def _dfix(d):
                            sl = pl.ds(bq * LANES, LANES)
                            dh, dl = d // DL, d % DL
                            pez = plsc.load_gather(
                                pe_t, [jnp.full((LANES,), d, jnp.int32)])
                            ob[o][dh, dl, sl] = (
                                (ob[o][dh, dl, sl] - pez) * mf + pez)

                writeout(t, o, wait=False)

            @pl.when(s0 + 2 < NSG)
            def _next_gather():
                start_gather(sg + 2, p)

    for j in range(NO):
        writeout(NCHUNK - NO + j, (NCHUNK - NO + j) % NO, wait=True)


def kernel(x, table):
    xr = (x.T.reshape(MAX_TOK, NW, BBLK).transpose(1, 0, 2)
          .reshape(NW, NSG, SG * BBLK).astype(jnp.int32))
    out5 = _emb_lookup(xr, _pe_table(), table)
    return out5.transpose(2, 4, 0, 1, 3).reshape(BATCH, MAX_TOK, DIMS)
